# baseline (speedup 1.0000x reference)
import functools
import jax
import jax.numpy as jnp
from jax import lax
from jax.experimental import pallas as pl
from jax.experimental.pallas import tpu as pltpu
from jax.experimental.pallas import tpu_sc as plsc

N = 10000
D_FEAT = 128
HIDDEN = 64
OUT = 128
E = 320000

NC = 2
NS = 16
NW = NC * NS
CH = 80
NBUF = 5
CW = 8
K = 125
NV = K // NBUF
N_PAD = 10240
RPS = N_PAD // NS
ACC_ROWS = N_PAD


def _agg_body(table, edges, ones_hbm, zero_cnt,
              out_sum, out_cnt,
              src_v, dst_v, rows_v, ones_v, acc_s, cnt_s, gsem, ssem, stg):
    c = lax.axis_index("c")
    s = lax.axis_index("s")
    w = s * NC + c
    r0 = s * RPS

    pltpu.async_copy(edges.at[0, w], src_v, stg)
    pltpu.async_copy(edges.at[1, w], dst_v, stg)
    pltpu.async_copy(ones_hbm, ones_v, stg)
    pltpu.async_copy(zero_cnt, cnt_s.at[pl.ds(r0, RPS)], stg)

    zv = jnp.zeros((16,), jnp.float32)

    def zfill(i, carry):
        for c4 in range(HIDDEN // 16):
            rows_v[0, 0, i, pl.ds(c4 * 16, 16)] = zv
        return carry

    lax.fori_loop(0, CH, zfill, 0)
    for t in range(RPS // CH):
        pltpu.sync_copy(rows_v.at[0, 0], acc_s.at[pl.ds(r0 + t * CH, CH)])

    pltpu.make_async_copy(edges.at[0, w], src_v, stg).wait()
    pltpu.make_async_copy(edges.at[1, w], dst_v, stg).wait()
    pltpu.make_async_copy(ones_hbm, ones_v, stg).wait()
    pltpu.make_async_copy(zero_cnt, cnt_s.at[pl.ds(r0, RPS)], stg).wait()

    def gat(j, p, b):
        return pltpu.make_async_copy(
            table.at[src_v.at[j]], rows_v.at[p, b], gsem.at[p, b])

    def sc_rows(j, p, b):
        return pltpu.make_async_copy(
            rows_v.at[p, b], acc_s.at[dst_v.at[j]], ssem.at[p, b])

    def sc_ones(j, p, b):
        return pltpu.make_async_copy(
            ones_v, cnt_s.at[dst_v.at[j]], ssem.at[p, b])

    for b in range(NBUF):
        gat(b, 0, b).start()

    plsc.subcore_barrier()

    def outer(g2, carry):
        for half in range(2):
            vi = 2 * g2 + half
            p = half
            q = 1 - half
            for b in range(NBUF):
                j = vi * NBUF + b
                gat(j, p, b).wait()
                pltpu.async_copy(rows_v.at[p, b], acc_s.at[dst_v.at[j]],
                                 ssem.at[p, b], add=True)
                pltpu.async_copy(ones_v, cnt_s.at[dst_v.at[j]],
                                 ssem.at[p, b], add=True)
            for b in range(NBUF):
                j2 = (vi + 1) * NBUF + b

                @pl.when(vi >= 1)
                def _(b=b, p=p, q=q):
                    sc_rows(0, q, b).wait()
                    sc_ones(0, q, b).wait()
                gat(j2, q, b).start()
        return carry

    lax.fori_loop(0, (NV - 1) // 2, outer, 0)

    for b in range(NBUF):
        j = (NV - 1) * NBUF + b
        gat(j, 0, b).wait()
        pltpu.async_copy(rows_v.at[0, b], acc_s.at[dst_v.at[j]],
                         ssem.at[0, b], add=True)
        pltpu.async_copy(ones_v, cnt_s.at[dst_v.at[j]],
                         ssem.at[0, b], add=True)

    for p in range(2):
        for b in range(NBUF):
            sc_rows(0, p, b).wait()
            sc_ones(0, p, b).wait()

    plsc.subcore_barrier()

    pltpu.sync_copy(acc_s.at[pl.ds(r0, RPS)], out_sum.at[c, pl.ds(r0, RPS)])
    pltpu.sync_copy(cnt_s.at[pl.ds(r0, RPS)], out_cnt.at[c, pl.ds(r0, RPS)])


_agg = functools.partial(
    pl.kernel,
    mesh=plsc.VectorSubcoreMesh(core_axis_name="c", subcore_axis_name="s"),
    out_type=[
        jax.ShapeDtypeStruct((NC, N_PAD, HIDDEN), jnp.float32),
        jax.ShapeDtypeStruct((NC, N_PAD, CW), jnp.float32),
    ],
    scratch_types=[
        pltpu.VMEM((K, CH), jnp.int32),
        pltpu.VMEM((K, CH), jnp.int32),
        pltpu.VMEM((2, NBUF, CH, HIDDEN), jnp.float32),
        pltpu.VMEM((CH, CW), jnp.float32),
        pltpu.VMEM_SHARED((ACC_ROWS, HIDDEN), jnp.float32),
        pltpu.VMEM_SHARED((ACC_ROWS, CW), jnp.float32),
        pltpu.SemaphoreType.DMA((2, NBUF)),
        pltpu.SemaphoreType.DMA((2, NBUF)),
        pltpu.SemaphoreType.DMA,
    ],
    compiler_params=pltpu.CompilerParams(use_tc_tiling_on_sc=False),
)(_agg_body)


ROWB = 1000


def _pre_body(x_ref, wl_ref, wr_ref, b_ref, y_ref, z_ref):
    xb = x_ref[...]
    y_ref[...] = jnp.dot(xb, wl_ref[...], preferred_element_type=jnp.float32)
    z_ref[...] = (jnp.dot(xb, wr_ref[...], preferred_element_type=jnp.float32)
                  + b_ref[...])


def _mid_body(s_ref_a, s_ref_b, c_ref_a, c_ref_b, z0_ref, h_ref):
    cnt = c_ref_a[0] + c_ref_b[0]
    inv = 1.0 / jnp.maximum(cnt[:, 0:1], 1.0)
    h_ref[...] = jnp.maximum(
        (s_ref_a[0] + s_ref_b[0]) * inv + z0_ref[...], 0.0)


def _z1_body(h_ref, wr_ref, b_ref, z1_ref):
    z1_ref[...] = (jnp.dot(h_ref[...], wr_ref[...],
                           preferred_element_type=jnp.float32) + b_ref[...])


def _post_body(s_ref_a, s_ref_b, c_ref_a, c_ref_b, z1_ref, wl_ref, o_ref):
    cnt = c_ref_a[0] + c_ref_b[0]
    inv = 1.0 / jnp.maximum(cnt[:, 0:1], 1.0)
    m = (s_ref_a[0] + s_ref_b[0]) * inv
    o_ref[...] = (jnp.dot(m, wl_ref[...], preferred_element_type=jnp.float32)
                  + z1_ref[...])


def _row_spec(w):
    return pl.BlockSpec((ROWB, w), lambda i: (i, 0))


def _rep_spec(h, w):
    return pl.BlockSpec((h, w), lambda i: (0, 0))


def _part_spec(w, core):
    return pl.BlockSpec((1, ROWB, w), lambda i, c=core: (c, i, 0))


_pre = pl.pallas_call(
    _pre_body,
    grid=(N // ROWB,),
    in_specs=[_row_spec(D_FEAT), _rep_spec(D_FEAT, HIDDEN),
              _rep_spec(D_FEAT, HIDDEN), _rep_spec(1, HIDDEN)],
    out_specs=[_row_spec(HIDDEN), _row_spec(HIDDEN)],
    out_shape=[jax.ShapeDtypeStruct((N, HIDDEN), jnp.float32),
               jax.ShapeDtypeStruct((N, HIDDEN), jnp.float32)],
)

_mid = pl.pallas_call(
    _mid_body,
    grid=(N // ROWB,),
    in_specs=[_part_spec(HIDDEN, 0), _part_spec(HIDDEN, 1),
              _part_spec(CW, 0), _part_spec(CW, 1),
              _row_spec(HIDDEN)],
    out_specs=_row_spec(HIDDEN),
    out_shape=jax.ShapeDtypeStruct((N, HIDDEN), jnp.float32),
)

_z1 = pl.pallas_call(
    _z1_body,
    grid=(N // ROWB,),
    in_specs=[_row_spec(HIDDEN), _rep_spec(HIDDEN, OUT), _rep_spec(1, OUT)],
    out_specs=_row_spec(OUT),
    out_shape=jax.ShapeDtypeStruct((N, OUT), jnp.float32),
)

_post = pl.pallas_call(
    _post_body,
    grid=(N // ROWB,),
    in_specs=[_part_spec(HIDDEN, 0), _part_spec(HIDDEN, 1),
              _part_spec(CW, 0), _part_spec(CW, 1),
              _row_spec(OUT), _rep_spec(HIDDEN, OUT)],
    out_specs=_row_spec(OUT),
    out_shape=jax.ShapeDtypeStruct((N, OUT), jnp.float32),
)


def kernel(x, edge_index0, edge_index1, Wl0, Wr0, b0, Wl1, Wr1, b1):
    e0 = edge_index0.reshape(2, NW, K, CH)
    e1 = edge_index1.reshape(2, NW, K, CH)
    ones = jnp.ones((CH, CW), jnp.float32)
    zero_cnt = jnp.zeros((RPS, CW), jnp.float32)

    y0, z0 = _pre(x, Wl0, Wr0, b0.reshape(1, HIDDEN))
    sum0, cnt0 = _agg(y0, e0, ones, zero_cnt)
    h = _mid(sum0, sum0, cnt0, cnt0, z0)
    sum1, cnt1 = _agg(h, e1, ones, zero_cnt)
    z1 = _z1(h, Wr1, b1.reshape(1, OUT))
    out = _post(sum1, sum1, cnt1, cnt1, z1, Wl1)
    return out


# speedup vs baseline: 1.2469x; 1.2469x over previous
import functools
import jax
import jax.numpy as jnp
from jax import lax
from jax.experimental import pallas as pl
from jax.experimental.pallas import tpu as pltpu
from jax.experimental.pallas import tpu_sc as plsc

N = 10000
D_FEAT = 128
HIDDEN = 64
OUT = 128
E = 320000

NC = 2
NS = 16
NW = NC * NS
CH = 80
NBUF = 5
CW = 8
K = 125
NV = K // NBUF
N_PAD = 10240
RPS = N_PAD // NS
ACC_ROWS = N_PAD


def _agg_body(table, edges, ones_hbm, zero_cnt,
              out_sum, out_cnt,
              src_v, dst_v, rows_v, ones_v, acc_s, cnt_s, gsem, ssem, stg):
    c = lax.axis_index("c")
    s = lax.axis_index("s")
    w = s * NC + c
    r0 = s * RPS

    pltpu.async_copy(edges.at[0, w], src_v, stg)
    pltpu.async_copy(edges.at[1, w], dst_v, stg)
    pltpu.async_copy(ones_hbm, ones_v, stg)
    pltpu.async_copy(zero_cnt, cnt_s.at[pl.ds(r0, RPS)], stg)

    zv = jnp.zeros((32,), jnp.bfloat16)

    def zfill(i, carry):
        for c2 in range(HIDDEN // 32):
            rows_v[0, 0, i, pl.ds(c2 * 32, 32)] = zv
        return carry

    lax.fori_loop(0, CH, zfill, 0)
    for t in range(RPS // CH):
        pltpu.sync_copy(rows_v.at[0, 0], acc_s.at[pl.ds(r0 + t * CH, CH)])

    pltpu.make_async_copy(edges.at[0, w], src_v, stg).wait()
    pltpu.make_async_copy(edges.at[1, w], dst_v, stg).wait()
    pltpu.make_async_copy(ones_hbm, ones_v, stg).wait()
    pltpu.make_async_copy(zero_cnt, cnt_s.at[pl.ds(r0, RPS)], stg).wait()

    def gat(j, p, b):
        return pltpu.make_async_copy(
            table.at[src_v.at[j]], rows_v.at[p, b], gsem.at[p, b])

    def sc_rows(j, p, b):
        return pltpu.make_async_copy(
            rows_v.at[p, b], acc_s.at[dst_v.at[j]], ssem.at[p, b])

    def sc_ones(j, p, b):
        return pltpu.make_async_copy(
            ones_v, cnt_s.at[dst_v.at[j]], ssem.at[p, b])

    for b in range(NBUF):
        gat(b, 0, b).start()

    plsc.subcore_barrier()

    def outer(g2, carry):
        for half in range(2):
            vi = 2 * g2 + half
            p = half
            q = 1 - half
            for b in range(NBUF):
                j = vi * NBUF + b
                gat(j, p, b).wait()
                pltpu.async_copy(rows_v.at[p, b], acc_s.at[dst_v.at[j]],
                                 ssem.at[p, b], add=True)
                pltpu.async_copy(ones_v, cnt_s.at[dst_v.at[j]],
                                 ssem.at[p, b], add=True)
            for b in range(NBUF):
                j2 = (vi + 1) * NBUF + b

                @pl.when(vi >= 1)
                def _(b=b, p=p, q=q):
                    sc_rows(0, q, b).wait()
                    sc_ones(0, q, b).wait()
                gat(j2, q, b).start()
        return carry

    lax.fori_loop(0, (NV - 1) // 2, outer, 0)

    for b in range(NBUF):
        j = (NV - 1) * NBUF + b
        gat(j, 0, b).wait()
        pltpu.async_copy(rows_v.at[0, b], acc_s.at[dst_v.at[j]],
                         ssem.at[0, b], add=True)
        pltpu.async_copy(ones_v, cnt_s.at[dst_v.at[j]],
                         ssem.at[0, b], add=True)

    for p in range(2):
        for b in range(NBUF):
            sc_rows(0, p, b).wait()
            sc_ones(0, p, b).wait()

    plsc.subcore_barrier()

    pltpu.sync_copy(acc_s.at[pl.ds(r0, RPS)], out_sum.at[c, pl.ds(r0, RPS)])
    pltpu.sync_copy(cnt_s.at[pl.ds(r0, RPS)], out_cnt.at[c, pl.ds(r0, RPS)])


_agg = functools.partial(
    pl.kernel,
    mesh=plsc.VectorSubcoreMesh(core_axis_name="c", subcore_axis_name="s"),
    out_type=[
        jax.ShapeDtypeStruct((NC, N_PAD, HIDDEN), jnp.bfloat16),
        jax.ShapeDtypeStruct((NC, N_PAD, CW), jnp.float32),
    ],
    scratch_types=[
        pltpu.VMEM((K, CH), jnp.int32),
        pltpu.VMEM((K, CH), jnp.int32),
        pltpu.VMEM((2, NBUF, CH, HIDDEN), jnp.bfloat16),
        pltpu.VMEM((CH, CW), jnp.float32),
        pltpu.VMEM_SHARED((ACC_ROWS, HIDDEN), jnp.bfloat16),
        pltpu.VMEM_SHARED((ACC_ROWS, CW), jnp.float32),
        pltpu.SemaphoreType.DMA((2, NBUF)),
        pltpu.SemaphoreType.DMA((2, NBUF)),
        pltpu.SemaphoreType.DMA,
    ],
    compiler_params=pltpu.CompilerParams(use_tc_tiling_on_sc=False),
)(_agg_body)


ROWB = 1000


def _pre_body(x_ref, wl_ref, wr_ref, b_ref, y_ref, z_ref):
    xb = x_ref[...]
    y_ref[...] = jnp.dot(
        xb, wl_ref[...],
        preferred_element_type=jnp.float32).astype(jnp.bfloat16)
    z_ref[...] = (jnp.dot(xb, wr_ref[...], preferred_element_type=jnp.float32)
                  + b_ref[...])


def _mid_body(s_ref_a, s_ref_b, c_ref_a, c_ref_b, z0_ref, wr_ref, b_ref,
              h_ref, z1_ref):
    cnt = c_ref_a[0] + c_ref_b[0]
    inv = 1.0 / jnp.maximum(cnt[:, 0:1], 1.0)
    ssum = s_ref_a[0].astype(jnp.float32) + s_ref_b[0].astype(jnp.float32)
    h = jnp.maximum(ssum * inv + z0_ref[...], 0.0)
    h_ref[...] = h.astype(jnp.bfloat16)
    z1_ref[...] = (jnp.dot(h, wr_ref[...], preferred_element_type=jnp.float32)
                   + b_ref[...])


def _post_body(s_ref_a, s_ref_b, c_ref_a, c_ref_b, z1_ref, wl_ref, o_ref):
    cnt = c_ref_a[0] + c_ref_b[0]
    inv = 1.0 / jnp.maximum(cnt[:, 0:1], 1.0)
    ssum = s_ref_a[0].astype(jnp.float32) + s_ref_b[0].astype(jnp.float32)
    m = ssum * inv
    o_ref[...] = (jnp.dot(m, wl_ref[...], preferred_element_type=jnp.float32)
                  + z1_ref[...])


def _row_spec(w):
    return pl.BlockSpec((ROWB, w), lambda i: (i, 0))


def _rep_spec(h, w):
    return pl.BlockSpec((h, w), lambda i: (0, 0))


def _part_spec(w, core):
    return pl.BlockSpec((1, ROWB, w), lambda i, c=core: (c, i, 0))


_pre = pl.pallas_call(
    _pre_body,
    grid=(N // ROWB,),
    in_specs=[_row_spec(D_FEAT), _rep_spec(D_FEAT, HIDDEN),
              _rep_spec(D_FEAT, HIDDEN), _rep_spec(1, HIDDEN)],
    out_specs=[_row_spec(HIDDEN), _row_spec(HIDDEN)],
    out_shape=[jax.ShapeDtypeStruct((N, HIDDEN), jnp.bfloat16),
               jax.ShapeDtypeStruct((N, HIDDEN), jnp.float32)],
)

_mid = pl.pallas_call(
    _mid_body,
    grid=(N // ROWB,),
    in_specs=[_part_spec(HIDDEN, 0), _part_spec(HIDDEN, 1),
              _part_spec(CW, 0), _part_spec(CW, 1),
              _row_spec(HIDDEN), _rep_spec(HIDDEN, OUT), _rep_spec(1, OUT)],
    out_specs=[_row_spec(HIDDEN), _row_spec(OUT)],
    out_shape=[jax.ShapeDtypeStruct((N, HIDDEN), jnp.bfloat16),
               jax.ShapeDtypeStruct((N, OUT), jnp.float32)],
)

_post = pl.pallas_call(
    _post_body,
    grid=(N // ROWB,),
    in_specs=[_part_spec(HIDDEN, 0), _part_spec(HIDDEN, 1),
              _part_spec(CW, 0), _part_spec(CW, 1),
              _row_spec(OUT), _rep_spec(HIDDEN, OUT)],
    out_specs=_row_spec(OUT),
    out_shape=jax.ShapeDtypeStruct((N, OUT), jnp.float32),
)


def kernel(x, edge_index0, edge_index1, Wl0, Wr0, b0, Wl1, Wr1, b1):
    e0 = edge_index0.reshape(2, NW, K, CH)
    e1 = edge_index1.reshape(2, NW, K, CH)
    ones = jnp.ones((CH, CW), jnp.float32)
    zero_cnt = jnp.zeros((RPS, CW), jnp.float32)

    y0, z0 = _pre(x, Wl0, Wr0, b0.reshape(1, HIDDEN))
    sum0, cnt0 = _agg(y0, e0, ones, zero_cnt)
    h, z1 = _mid(sum0, sum0, cnt0, cnt0, z0, Wr1, b1.reshape(1, OUT))
    sum1, cnt1 = _agg(h, e1, ones, zero_cnt)
    out = _post(sum1, sum1, cnt1, cnt1, z1, Wl1)
    return out


# speedup vs baseline: 1.2711x; 1.0194x over previous
import functools
import jax
import jax.numpy as jnp
from jax import lax
from jax.experimental import pallas as pl
from jax.experimental.pallas import tpu as pltpu
from jax.experimental.pallas import tpu_sc as plsc

N = 10000
D_FEAT = 128
HIDDEN = 64
OUT = 128
E = 320000

NC = 2
NS = 16
NW = NC * NS
CH = 80
NBUF = 5
CW = 8
K = 125
NV = K // NBUF
N_PAD = 10240
RPS = N_PAD // NS
ACC_ROWS = N_PAD


def _agg_body(table, edges, ones_hbm, zero_cnt,
              out_sum, out_cnt,
              src_v, dst_v, rows_v, ones_v, acc_s, cnt_s, gsem, ssem, stg):
    c = lax.axis_index("c")
    s = lax.axis_index("s")
    w = s * NC + c
    r0 = s * RPS

    pltpu.async_copy(edges.at[0, w], src_v, stg)
    pltpu.async_copy(edges.at[1, w], dst_v, stg)
    pltpu.async_copy(ones_hbm, ones_v, stg)
    pltpu.async_copy(zero_cnt, cnt_s.at[pl.ds(r0, RPS)], stg)

    zv = jnp.zeros((32,), jnp.bfloat16)

    def zfill(i, carry):
        for c2 in range(HIDDEN // 32):
            rows_v[0, 0, i, pl.ds(c2 * 32, 32)] = zv
        return carry

    lax.fori_loop(0, CH, zfill, 0)
    for t in range(RPS // CH):
        pltpu.sync_copy(rows_v.at[0, 0], acc_s.at[pl.ds(r0 + t * CH, CH)])

    pltpu.make_async_copy(edges.at[0, w], src_v, stg).wait()
    pltpu.make_async_copy(edges.at[1, w], dst_v, stg).wait()
    pltpu.make_async_copy(ones_hbm, ones_v, stg).wait()
    pltpu.make_async_copy(zero_cnt, cnt_s.at[pl.ds(r0, RPS)], stg).wait()

    def gat(j, p, b):
        return pltpu.make_async_copy(
            table.at[src_v.at[j]], rows_v.at[p, b], gsem.at[p, b])

    def sc_rows(j, p, b):
        return pltpu.make_async_copy(
            rows_v.at[p, b], acc_s.at[dst_v.at[j]], ssem.at[p, b])

    def sc_ones(j, p, b):
        return pltpu.make_async_copy(
            ones_v, cnt_s.at[dst_v.at[j]], ssem.at[p, b])

    for b in range(NBUF):
        gat(b, 0, b).start()

    plsc.subcore_barrier()

    def outer(g2, carry):
        for half in range(2):
            vi = 2 * g2 + half
            p = half
            q = 1 - half
            for b in range(NBUF):
                j = vi * NBUF + b
                gat(j, p, b).wait()
                pltpu.async_copy(rows_v.at[p, b], acc_s.at[dst_v.at[j]],
                                 ssem.at[p, b], add=True)
                pltpu.async_copy(ones_v, cnt_s.at[dst_v.at[j]],
                                 ssem.at[p, b], add=True)
            for b in range(NBUF):
                j2 = (vi + 1) * NBUF + b

                @pl.when(vi >= 1)
                def _(b=b, p=p, q=q):
                    sc_rows(0, q, b).wait()
                    sc_ones(0, q, b).wait()
                gat(j2, q, b).start()
        return carry

    lax.fori_loop(0, (NV - 1) // 2, outer, 0)

    for b in range(NBUF):
        j = (NV - 1) * NBUF + b
        gat(j, 0, b).wait()
        pltpu.async_copy(rows_v.at[0, b], acc_s.at[dst_v.at[j]],
                         ssem.at[0, b], add=True)
        pltpu.async_copy(ones_v, cnt_s.at[dst_v.at[j]],
                         ssem.at[0, b], add=True)

    for p in range(2):
        for b in range(NBUF):
            sc_rows(0, p, b).wait()
            sc_ones(0, p, b).wait()

    plsc.subcore_barrier()

    pltpu.sync_copy(acc_s.at[pl.ds(r0, RPS)], out_sum.at[c, pl.ds(r0, RPS)])
    pltpu.sync_copy(cnt_s.at[pl.ds(r0, RPS)], out_cnt.at[c, pl.ds(r0, RPS)])


_agg = functools.partial(
    pl.kernel,
    mesh=plsc.VectorSubcoreMesh(core_axis_name="c", subcore_axis_name="s"),
    out_type=[
        jax.ShapeDtypeStruct((NC, N_PAD, HIDDEN), jnp.bfloat16),
        jax.ShapeDtypeStruct((NC, N_PAD, CW), jnp.float32),
    ],
    scratch_types=[
        pltpu.VMEM((K, CH), jnp.int32),
        pltpu.VMEM((K, CH), jnp.int32),
        pltpu.VMEM((2, NBUF, CH, HIDDEN), jnp.bfloat16),
        pltpu.VMEM((CH, CW), jnp.float32),
        pltpu.VMEM_SHARED((ACC_ROWS, HIDDEN), jnp.bfloat16),
        pltpu.VMEM_SHARED((ACC_ROWS, CW), jnp.float32),
        pltpu.SemaphoreType.DMA((2, NBUF)),
        pltpu.SemaphoreType.DMA((2, NBUF)),
        pltpu.SemaphoreType.DMA,
    ],
    compiler_params=pltpu.CompilerParams(use_tc_tiling_on_sc=False),
)(_agg_body)


ROWB = 1000


def _pre_body(x_ref, wl_ref, wr_ref, b_ref, y_ref, z_ref):
    xb = x_ref[...]
    y_ref[...] = jnp.dot(
        xb, wl_ref[...],
        preferred_element_type=jnp.float32).astype(jnp.bfloat16)
    z_ref[...] = (jnp.dot(xb, wr_ref[...], preferred_element_type=jnp.float32)
                  + b_ref[...]).astype(jnp.bfloat16)


def _mid_body(s_ref_a, s_ref_b, c_ref_a, c_ref_b, z0_ref, wr_ref, b_ref,
              h_ref, z1_ref):
    cnt = c_ref_a[0] + c_ref_b[0]
    inv = 1.0 / jnp.maximum(cnt[:, 0:1], 1.0)
    ssum = s_ref_a[0].astype(jnp.float32) + s_ref_b[0].astype(jnp.float32)
    h = jnp.maximum(ssum * inv + z0_ref[...].astype(jnp.float32), 0.0)
    h_ref[...] = h.astype(jnp.bfloat16)
    z1_ref[...] = (jnp.dot(h, wr_ref[...], preferred_element_type=jnp.float32)
                   + b_ref[...]).astype(jnp.bfloat16)


def _post_body(s_ref_a, s_ref_b, c_ref_a, c_ref_b, z1_ref, wl_ref, o_ref):
    cnt = c_ref_a[0] + c_ref_b[0]
    inv = 1.0 / jnp.maximum(cnt[:, 0:1], 1.0)
    ssum = s_ref_a[0].astype(jnp.float32) + s_ref_b[0].astype(jnp.float32)
    m = ssum * inv
    o_ref[...] = (jnp.dot(m, wl_ref[...], preferred_element_type=jnp.float32)
                  + z1_ref[...].astype(jnp.float32))


def _row_spec(w):
    return pl.BlockSpec((ROWB, w), lambda i: (i, 0))


def _rep_spec(h, w):
    return pl.BlockSpec((h, w), lambda i: (0, 0))


def _part_spec(w, core):
    return pl.BlockSpec((1, ROWB, w), lambda i, c=core: (c, i, 0))


_pre = pl.pallas_call(
    _pre_body,
    grid=(N // ROWB,),
    in_specs=[_row_spec(D_FEAT), _rep_spec(D_FEAT, HIDDEN),
              _rep_spec(D_FEAT, HIDDEN), _rep_spec(1, HIDDEN)],
    out_specs=[_row_spec(HIDDEN), _row_spec(HIDDEN)],
    out_shape=[jax.ShapeDtypeStruct((N, HIDDEN), jnp.bfloat16),
               jax.ShapeDtypeStruct((N, HIDDEN), jnp.bfloat16)],
)

_mid = pl.pallas_call(
    _mid_body,
    grid=(N // ROWB,),
    in_specs=[_part_spec(HIDDEN, 0), _part_spec(HIDDEN, 1),
              _part_spec(CW, 0), _part_spec(CW, 1),
              _row_spec(HIDDEN), _rep_spec(HIDDEN, OUT), _rep_spec(1, OUT)],
    out_specs=[_row_spec(HIDDEN), _row_spec(OUT)],
    out_shape=[jax.ShapeDtypeStruct((N, HIDDEN), jnp.bfloat16),
               jax.ShapeDtypeStruct((N, OUT), jnp.bfloat16)],
)

_post = pl.pallas_call(
    _post_body,
    grid=(N // ROWB,),
    in_specs=[_part_spec(HIDDEN, 0), _part_spec(HIDDEN, 1),
              _part_spec(CW, 0), _part_spec(CW, 1),
              _row_spec(OUT), _rep_spec(HIDDEN, OUT)],
    out_specs=_row_spec(OUT),
    out_shape=jax.ShapeDtypeStruct((N, OUT), jnp.float32),
)


def kernel(x, edge_index0, edge_index1, Wl0, Wr0, b0, Wl1, Wr1, b1):
    e0 = edge_index0.reshape(2, NW, K, CH)
    e1 = edge_index1.reshape(2, NW, K, CH)
    ones = jnp.ones((CH, CW), jnp.float32)
    zero_cnt = jnp.zeros((RPS, CW), jnp.float32)

    y0, z0 = _pre(x, Wl0, Wr0, b0.reshape(1, HIDDEN))
    sum0, cnt0 = _agg(y0, e0, ones, zero_cnt)
    h, z1 = _mid(sum0, sum0, cnt0, cnt0, z0, Wr1, b1.reshape(1, OUT))
    sum1, cnt1 = _agg(h, e1, ones, zero_cnt)
    out = _post(sum1, sum1, cnt1, cnt1, z1, Wl1)
    return out


# speedup vs baseline: 1.3168x; 1.0360x over previous
import functools
import jax
import jax.numpy as jnp
from jax import lax
from jax.experimental import pallas as pl
from jax.experimental.pallas import tpu as pltpu
from jax.experimental.pallas import tpu_sc as plsc

N = 10000
D_FEAT = 128
HIDDEN = 64
OUT = 128
E = 320000

NC = 2
NS = 16
NW = NC * NS
CH = 80
NBUF = 5
CW = 8
K = 125
NV = K // NBUF
N_PAD = 10240
RPS = N_PAD // NS
ACC_ROWS = N_PAD


def _cnt_body(e0, e1, ones_hbm, zero_cnt,
              out_cnt0, out_cnt1,
              dst_v, ones_v, cnt0_s, cnt1_s, ssem, stg):
    c = lax.axis_index("c")
    s = lax.axis_index("s")
    w = s * NC + c
    r0 = s * RPS

    pltpu.async_copy(ones_hbm, ones_v, stg)
    pltpu.async_copy(zero_cnt, cnt0_s.at[pl.ds(r0, RPS)], stg)
    pltpu.async_copy(zero_cnt, cnt1_s.at[pl.ds(r0, RPS)], stg)
    pltpu.sync_copy(e0.at[1, w], dst_v.at[0])
    pltpu.sync_copy(e1.at[1, w], dst_v.at[1])
    pltpu.make_async_copy(ones_hbm, ones_v, stg).wait()
    pltpu.make_async_copy(zero_cnt, cnt0_s.at[pl.ds(r0, RPS)], stg).wait()
    pltpu.make_async_copy(zero_cnt, cnt1_s.at[pl.ds(r0, RPS)], stg).wait()

    plsc.subcore_barrier()

    def ring(layer, tab):
        def body(j, carry):
            pltpu.async_copy(ones_v, tab.at[dst_v.at[layer, j]], ssem,
                             add=True)

            @pl.when(j >= 8)
            def _():
                pltpu.make_async_copy(
                    ones_v, tab.at[dst_v.at[layer, 0]], ssem).wait()
            return carry

        lax.fori_loop(0, K, body, 0)
        for _ in range(8):
            pltpu.make_async_copy(
                ones_v, tab.at[dst_v.at[layer, 0]], ssem).wait()

    ring(0, cnt0_s)
    ring(1, cnt1_s)

    plsc.subcore_barrier()

    pltpu.sync_copy(cnt0_s.at[pl.ds(r0, RPS)], out_cnt0.at[c, pl.ds(r0, RPS)])
    pltpu.sync_copy(cnt1_s.at[pl.ds(r0, RPS)], out_cnt1.at[c, pl.ds(r0, RPS)])


_cnt = functools.partial(
    pl.kernel,
    mesh=plsc.VectorSubcoreMesh(core_axis_name="c", subcore_axis_name="s"),
    out_type=[
        jax.ShapeDtypeStruct((NC, N_PAD, CW), jnp.float32),
        jax.ShapeDtypeStruct((NC, N_PAD, CW), jnp.float32),
    ],
    scratch_types=[
        pltpu.VMEM((2, K, CH), jnp.int32),
        pltpu.VMEM((CH, CW), jnp.float32),
        pltpu.VMEM_SHARED((ACC_ROWS, CW), jnp.float32),
        pltpu.VMEM_SHARED((ACC_ROWS, CW), jnp.float32),
        pltpu.SemaphoreType.DMA,
        pltpu.SemaphoreType.DMA,
    ],
    compiler_params=pltpu.CompilerParams(use_tc_tiling_on_sc=False),
)(_cnt_body)


def _agg_body(table, edges,
              out_sum,
              src_v, dst_v, rows_v, acc_s, gsem, ssem, stg):
    c = lax.axis_index("c")
    s = lax.axis_index("s")
    w = s * NC + c
    r0 = s * RPS

    pltpu.async_copy(edges.at[0, w], src_v, stg)
    pltpu.async_copy(edges.at[1, w], dst_v, stg)

    zv = jnp.zeros((32,), jnp.bfloat16)

    def zfill(i, carry):
        for c2 in range(HIDDEN // 32):
            rows_v[0, 0, i, pl.ds(c2 * 32, 32)] = zv
        return carry

    lax.fori_loop(0, CH, zfill, 0)
    for t in range(RPS // CH):
        pltpu.sync_copy(rows_v.at[0, 0], acc_s.at[pl.ds(r0 + t * CH, CH)])

    pltpu.make_async_copy(edges.at[0, w], src_v, stg).wait()
    pltpu.make_async_copy(edges.at[1, w], dst_v, stg).wait()

    def gat(j, p, b):
        return pltpu.make_async_copy(
            table.at[src_v.at[j]], rows_v.at[p, b], gsem.at[p, b])

    def sc_rows(j, p, b):
        return pltpu.make_async_copy(
            rows_v.at[p, b], acc_s.at[dst_v.at[j]], ssem.at[p, b])

    for b in range(NBUF):
        gat(b, 0, b).start()

    plsc.subcore_barrier()

    def outer(g2, carry):
        for half in range(2):
            vi = 2 * g2 + half
            p = half
            q = 1 - half
            for b in range(NBUF):
                j = vi * NBUF + b
                gat(j, p, b).wait()
                pltpu.async_copy(rows_v.at[p, b], acc_s.at[dst_v.at[j]],
                                 ssem.at[p, b], add=True)
            for b in range(NBUF):
                j2 = (vi + 1) * NBUF + b

                @pl.when(vi >= 1)
                def _(b=b, q=q):
                    sc_rows(0, q, b).wait()
                gat(j2, q, b).start()
        return carry

    lax.fori_loop(0, (NV - 1) // 2, outer, 0)

    for b in range(NBUF):
        j = (NV - 1) * NBUF + b
        gat(j, 0, b).wait()
        pltpu.async_copy(rows_v.at[0, b], acc_s.at[dst_v.at[j]],
                         ssem.at[0, b], add=True)

    for p in range(2):
        for b in range(NBUF):
            sc_rows(0, p, b).wait()

    plsc.subcore_barrier()

    pltpu.sync_copy(acc_s.at[pl.ds(r0, RPS)], out_sum.at[c, pl.ds(r0, RPS)])


_agg = functools.partial(
    pl.kernel,
    mesh=plsc.VectorSubcoreMesh(core_axis_name="c", subcore_axis_name="s"),
    out_type=jax.ShapeDtypeStruct((NC, N_PAD, HIDDEN), jnp.bfloat16),
    scratch_types=[
        pltpu.VMEM((K, CH), jnp.int32),
        pltpu.VMEM((K, CH), jnp.int32),
        pltpu.VMEM((2, NBUF, CH, HIDDEN), jnp.bfloat16),
        pltpu.VMEM_SHARED((ACC_ROWS, HIDDEN), jnp.bfloat16),
        pltpu.SemaphoreType.DMA((2, NBUF)),
        pltpu.SemaphoreType.DMA((2, NBUF)),
        pltpu.SemaphoreType.DMA,
    ],
    compiler_params=pltpu.CompilerParams(use_tc_tiling_on_sc=False),
)(_agg_body)


ROWB = 1000


def _pre_body(x_ref, wl_ref, wr_ref, b_ref, y_ref, z_ref):
    xb = x_ref[...]
    y_ref[...] = jnp.dot(
        xb, wl_ref[...],
        preferred_element_type=jnp.float32).astype(jnp.bfloat16)
    z_ref[...] = (jnp.dot(xb, wr_ref[...], preferred_element_type=jnp.float32)
                  + b_ref[...]).astype(jnp.bfloat16)


def _mid_body(s_ref_a, s_ref_b, c_ref_a, c_ref_b, z0_ref, wr_ref, b_ref,
              h_ref, z1_ref):
    cnt = c_ref_a[0] + c_ref_b[0]
    inv = 1.0 / jnp.maximum(cnt[:, 0:1], 1.0)
    ssum = s_ref_a[0].astype(jnp.float32) + s_ref_b[0].astype(jnp.float32)
    h = jnp.maximum(ssum * inv + z0_ref[...].astype(jnp.float32), 0.0)
    h_ref[...] = h.astype(jnp.bfloat16)
    z1_ref[...] = (jnp.dot(h, wr_ref[...], preferred_element_type=jnp.float32)
                   + b_ref[...]).astype(jnp.bfloat16)


def _post_body(s_ref_a, s_ref_b, c_ref_a, c_ref_b, z1_ref, wl_ref, o_ref):
    cnt = c_ref_a[0] + c_ref_b[0]
    inv = 1.0 / jnp.maximum(cnt[:, 0:1], 1.0)
    ssum = s_ref_a[0].astype(jnp.float32) + s_ref_b[0].astype(jnp.float32)
    m = ssum * inv
    o_ref[...] = (jnp.dot(m, wl_ref[...], preferred_element_type=jnp.float32)
                  + z1_ref[...].astype(jnp.float32))


def _row_spec(w):
    return pl.BlockSpec((ROWB, w), lambda i: (i, 0))


def _rep_spec(h, w):
    return pl.BlockSpec((h, w), lambda i: (0, 0))


def _part_spec(w, core):
    return pl.BlockSpec((1, ROWB, w), lambda i, c=core: (c, i, 0))


_pre = pl.pallas_call(
    _pre_body,
    grid=(N // ROWB,),
    in_specs=[_row_spec(D_FEAT), _rep_spec(D_FEAT, HIDDEN),
              _rep_spec(D_FEAT, HIDDEN), _rep_spec(1, HIDDEN)],
    out_specs=[_row_spec(HIDDEN), _row_spec(HIDDEN)],
    out_shape=[jax.ShapeDtypeStruct((N, HIDDEN), jnp.bfloat16),
               jax.ShapeDtypeStruct((N, HIDDEN), jnp.bfloat16)],
)

_mid = pl.pallas_call(
    _mid_body,
    grid=(N // ROWB,),
    in_specs=[_part_spec(HIDDEN, 0), _part_spec(HIDDEN, 1),
              _part_spec(CW, 0), _part_spec(CW, 1),
              _row_spec(HIDDEN), _rep_spec(HIDDEN, OUT), _rep_spec(1, OUT)],
    out_specs=[_row_spec(HIDDEN), _row_spec(OUT)],
    out_shape=[jax.ShapeDtypeStruct((N, HIDDEN), jnp.bfloat16),
               jax.ShapeDtypeStruct((N, OUT), jnp.bfloat16)],
)

_post = pl.pallas_call(
    _post_body,
    grid=(N // ROWB,),
    in_specs=[_part_spec(HIDDEN, 0), _part_spec(HIDDEN, 1),
              _part_spec(CW, 0), _part_spec(CW, 1),
              _row_spec(OUT), _rep_spec(HIDDEN, OUT)],
    out_specs=_row_spec(OUT),
    out_shape=jax.ShapeDtypeStruct((N, OUT), jnp.float32),
)


def kernel(x, edge_index0, edge_index1, Wl0, Wr0, b0, Wl1, Wr1, b1):
    e0 = edge_index0.reshape(2, NW, K, CH)
    e1 = edge_index1.reshape(2, NW, K, CH)
    ones = jnp.ones((CH, CW), jnp.float32)
    zero_cnt = jnp.zeros((RPS, CW), jnp.float32)

    cnt0, cnt1 = _cnt(e0, e1, ones, zero_cnt)
    y0, z0 = _pre(x, Wl0, Wr0, b0.reshape(1, HIDDEN))
    sum0 = _agg(y0, e0)
    h, z1 = _mid(sum0, sum0, cnt0, cnt0, z0, Wr1, b1.reshape(1, OUT))
    sum1 = _agg(h, e1)
    out = _post(sum1, sum1, cnt1, cnt1, z1, Wl1)
    return out
